# Initial kernel scaffold; baseline (speedup 1.0000x reference)
import jax
import jax.numpy as jnp
from jax.experimental import pallas as pl


def kernel(X, edge_index, edge_weight, concat_layers, H, C, Wxi, Whi, wci, b_i, Wxf, Whf, wcf, b_f, Wxc, Whc, b_c, Wxo, Who, wco, b_o, W_fc1, b_fc1, W_fc2, b_fc2, g_o, be_o, g_h, be_h, g_c, be_c):
    raise NotImplementedError("write your pallas kernel here")


# speedup vs baseline: 3.0019x; 3.0019x over previous
import functools

import jax
import jax.numpy as jnp
from jax.experimental import pallas as pl

N = 10000
F_IN = 128
HID = 128

_BB = 400


def _stage_b_body(p1_ref, x_ref, h0_ref, c0_ref, dinv_ref, catp_ref,
                  wz_ref, wci_ref, wcf_ref, wco_ref, bi_ref, bf_ref, bc_ref, bo_ref,
                  wfc1a_ref, wfc1b_ref, bfc1_ref,
                  gh_ref, beh_ref, gc_ref, bec_ref, go_ref, beo_ref,
                  hid_ref, cell_ref, u1_ref, u1s_ref):
    dinv = dinv_ref[...]
    x = x_ref[...]
    h0 = h0_ref[...]
    c0 = c0_ref[...]
    d2 = dinv * dinv
    za = jnp.concatenate(
        [dinv * p1_ref[:, :HID] + d2 * x,
         dinv * p1_ref[:, HID:] + d2 * h0], axis=1)
    g = jnp.dot(za, wz_ref[...], preferred_element_type=jnp.float32)
    I = jax.nn.sigmoid(g[:, 0 * HID:1 * HID] + wci_ref[...] * c0 + bi_ref[...])
    F = jax.nn.sigmoid(g[:, 1 * HID:2 * HID] + wcf_ref[...] * c0 + bf_ref[...])
    T = jnp.tanh(g[:, 2 * HID:3 * HID] + bc_ref[...])
    c_new = F * c0 + I * T
    O = jax.nn.sigmoid(g[:, 3 * HID:4 * HID] + wco_ref[...] * c_new + bo_ref[...])
    h_new = O * jnp.tanh(c_new)

    def ln(v, gg, bb):
        m = jnp.mean(v, axis=-1, keepdims=True)
        vv = jnp.mean((v - m) * (v - m), axis=-1, keepdims=True)
        return (v - m) * jax.lax.rsqrt(vv + 1e-5) * gg + bb

    hid_ref[...] = ln(h_new, gh_ref[...], beh_ref[...])
    cell_ref[...] = ln(c_new, gc_ref[...], bec_ref[...])
    out = jnp.maximum(ln(h_new, go_ref[...], beo_ref[...]), 0.0)
    u1 = (jnp.dot(out, wfc1a_ref[...], preferred_element_type=jnp.float32)
          + jnp.dot(catp_ref[...], wfc1b_ref[...], preferred_element_type=jnp.float32))
    u1_ref[...] = u1
    u1s_ref[...] = u1 * dinv


def _stage_a_body(deg_ref, x_ref, h0_ref, zs_ref, dinv_ref):
    deg = deg_ref[...] + 1.0
    dinv = jax.lax.rsqrt(deg)
    zs_ref[:, :HID] = x_ref[...] * dinv
    zs_ref[:, HID:] = h0_ref[...] * dinv
    dinv_ref[...] = dinv


def _stage_c_body(p2_ref, u1_ref, dinv_ref, bfc1_ref, w2_ref, u2_ref, u2s_ref):
    dinv = dinv_ref[...]
    a1 = dinv * p2_ref[...] + dinv * dinv * u1_ref[...] + bfc1_ref[...]
    r = jnp.maximum(a1, 0.0)
    u2 = jnp.sum(r * w2_ref[...], axis=-1, keepdims=True)
    u2_ref[...] = u2
    u2s_ref[...] = u2 * dinv


def _stage_d_body(p3_ref, u2_ref, dinv_ref, x0_ref, bfc2_ref, out_ref):
    dinv = dinv_ref[...]
    v = dinv * p3_ref[...] + dinv * dinv * u2_ref[...] + bfc2_ref[...]
    out_ref[...] = jnp.tanh(v) + x0_ref[...]


def _row_spec(w):
    return pl.BlockSpec((_BB, w), lambda i: (i, 0))


def _bcast_spec(shape):
    return pl.BlockSpec(shape, lambda i: tuple(0 for _ in shape))


def kernel(X, edge_index, edge_weight, concat_layers, H, C,
           Wxi, Whi, wci, b_i, Wxf, Whf, wcf, b_f, Wxc, Whc, b_c,
           Wxo, Who, wco, b_o, W_fc1, b_fc1, W_fc2, b_fc2,
           g_o, be_o, g_h, be_h, g_c, be_c):
    src = edge_index[0]
    dst = edge_index[1]
    h0 = H[0]
    c0 = C[0]
    grid = (N // _BB,)

    deg = jax.ops.segment_sum(edge_weight, dst, num_segments=N)[:, None]

    zs, dinv = pl.pallas_call(
        _stage_a_body,
        grid=grid,
        in_specs=[_row_spec(1), _row_spec(HID), _row_spec(HID)],
        out_specs=[_row_spec(2 * HID), _row_spec(1)],
        out_shape=[jax.ShapeDtypeStruct((N, 2 * HID), jnp.float32),
                   jax.ShapeDtypeStruct((N, 1), jnp.float32)],
    )(deg, X, h0)

    p1 = jax.ops.segment_sum(edge_weight[:, None] * zs[src], dst, num_segments=N)

    wz = jnp.concatenate([
        jnp.concatenate([Wxi, Wxf, Wxc, Wxo], axis=1),
        jnp.concatenate([Whi, Whf, Whc, Who], axis=1)], axis=0)
    catp = jnp.pad(concat_layers, ((0, 0), (0, 5)))
    wfc1a = W_fc1[:HID]
    wfc1b = jnp.pad(W_fc1[HID:], ((0, 5), (0, 0)))

    hidden, cell, u1, u1s = pl.pallas_call(
        _stage_b_body,
        grid=grid,
        in_specs=[_row_spec(2 * HID), _row_spec(HID), _row_spec(HID), _row_spec(HID),
                  _row_spec(1), _row_spec(8),
                  _bcast_spec((2 * HID, 4 * HID)),
                  _bcast_spec((1, HID)), _bcast_spec((1, HID)), _bcast_spec((1, HID)),
                  _bcast_spec((1, HID)), _bcast_spec((1, HID)), _bcast_spec((1, HID)),
                  _bcast_spec((1, HID)),
                  _bcast_spec((HID, HID)), _bcast_spec((8, HID)), _bcast_spec((1, HID)),
                  _bcast_spec((1, HID)), _bcast_spec((1, HID)), _bcast_spec((1, HID)),
                  _bcast_spec((1, HID)), _bcast_spec((1, HID)), _bcast_spec((1, HID))],
        out_specs=[_row_spec(HID)] * 4,
        out_shape=[jax.ShapeDtypeStruct((N, HID), jnp.float32)] * 4,
    )(p1, X, h0, c0, dinv, catp, wz,
      wci, wcf, wco, b_i, b_f, b_c, b_o,
      wfc1a, wfc1b, b_fc1[None, :],
      g_h[None, :], be_h[None, :], g_c[None, :], be_c[None, :],
      g_o[None, :], be_o[None, :])

    p2 = jax.ops.segment_sum(edge_weight[:, None] * u1s[src], dst, num_segments=N)

    u2, u2s = pl.pallas_call(
        _stage_c_body,
        grid=grid,
        in_specs=[_row_spec(HID), _row_spec(HID), _row_spec(1),
                  _bcast_spec((1, HID)), _bcast_spec((1, HID))],
        out_specs=[_row_spec(1), _row_spec(1)],
        out_shape=[jax.ShapeDtypeStruct((N, 1), jnp.float32)] * 2,
    )(p2, u1, dinv, b_fc1[None, :], W_fc2.T)

    p3 = jax.ops.segment_sum(edge_weight * u2s[src, 0], dst, num_segments=N)[:, None]

    out = pl.pallas_call(
        _stage_d_body,
        grid=grid,
        in_specs=[_row_spec(1), _row_spec(1), _row_spec(1), _row_spec(1),
                  _bcast_spec((1, 1))],
        out_specs=_row_spec(1),
        out_shape=jax.ShapeDtypeStruct((N, 1), jnp.float32),
    )(p3, u2, dinv, X[:, 0:1], b_fc2[None, :])

    return (out, hidden[None], cell[None])


# speedup vs baseline: 15.8270x; 5.2722x over previous
import functools

import jax
import jax.numpy as jnp
from jax import lax
from jax.experimental import pallas as pl
from jax.experimental.pallas import tpu as pltpu
from jax.experimental.pallas import tpu_sc as plsc

N = 10000
F_IN = 128
HID = 128
E = 320000

_NC, _NS, _L = 2, 16, 16
_EP = 327680
_NP = 10240
_CH = 1024
_SUB = _CH // 128

_BB = 400


def _sc_mesh():
    return plsc.VectorSubcoreMesh(core_axis_name="c", subcore_axis_name="s")


def _agg_wide_body(feature_split, src_r, dst_r, w_r, tab_r, zeros_r, out_r,
                   idxv, dstv, wv, rows, acc, sem):
    c = lax.axis_index("c")
    s = lax.axis_index("s")
    npr = _NP // _NS
    pltpu.sync_copy(zeros_r.at[pl.ds(s * npr, npr)], acc.at[pl.ds(s * npr, npr)])
    plsc.subcore_barrier()
    if feature_split:
        epw = _EP // _NS
        rr0 = s * (epw // _CH)
        coff = c * N
    else:
        epw = _EP // (_NC * _NS)
        wid = s * _NC + c
        rr0 = wid * (epw // _CH)
        coff = 0

    def chunk(i, car):
        rr = rr0 + i
        pltpu.sync_copy(src_r.at[rr], idxv)
        pltpu.sync_copy(dst_r.at[rr], dstv)
        pltpu.sync_copy(w_r.at[pl.ds(rr * _CH, _CH)], wv)
        if feature_split:
            for j in range(_SUB):
                for k in range(128 // _L):
                    sl = pl.ds(k * _L, _L)
                    idxv[j, sl] = idxv[j, sl] + coff
        for j in range(_SUB):
            pltpu.async_copy(tab_r.at[idxv.at[j]], rows, sem).wait()

            def scale(e, car2):
                wb = plsc.load_gather(wv, [jnp.broadcast_to(j * 128 + e, (_L,))])
                for q in range(128 // _L):
                    sl = pl.ds(q * _L, _L)
                    rows[e, sl] = rows[e, sl] * wb
                return car2

            lax.fori_loop(0, 128, scale, 0, unroll=2)
            pltpu.sync_copy(rows, acc.at[dstv.at[j]], add=True)
        return car

    lax.fori_loop(0, epw // _CH, chunk, 0)
    plsc.subcore_barrier()
    pltpu.sync_copy(acc.at[pl.ds(s * npr, npr)],
                    out_r.at[c, pl.ds(s * npr, npr)])


def _agg_wide(table, src3d, dst3d, wpad, zeros_w, feature_split):
    body = functools.partial(_agg_wide_body, feature_split)
    return pl.kernel(
        body,
        out_type=jax.ShapeDtypeStruct((_NC, _NP, 128), jnp.float32),
        mesh=_sc_mesh(),
        compiler_params=pltpu.CompilerParams(needs_layout_passes=False),
        scratch_types=[
            pltpu.VMEM((_SUB, 128), jnp.int32),
            pltpu.VMEM((_SUB, 128), jnp.int32),
            pltpu.VMEM((_CH,), jnp.float32),
            pltpu.VMEM((128, 128), jnp.float32),
            pltpu.VMEM_SHARED((_NP, 128), jnp.float32),
            pltpu.SemaphoreType.DMA,
        ],
    )(src3d, dst3d, wpad, table, zeros_w)


def _deg_body(dst_r, w_r, zeros_r, out_r, dstv, wv, acc):
    c = lax.axis_index("c")
    s = lax.axis_index("s")
    npr = _NP // _NS
    pltpu.sync_copy(zeros_r.at[pl.ds(s * npr, npr)], acc.at[pl.ds(s * npr, npr)])
    plsc.subcore_barrier()
    epw = _EP // (_NC * _NS)
    wid = s * _NC + c
    rr0 = wid * (epw // _CH)

    def chunk(i, car):
        rr = rr0 + i
        pltpu.sync_copy(dst_r.at[rr], dstv)
        pltpu.sync_copy(w_r.at[pl.ds(rr * _CH, _CH)], wv)
        for j in range(_SUB):
            pltpu.sync_copy(wv.at[pl.ds(j * 128, 128)],
                            acc.at[dstv.at[j]], add=True)
        return car

    lax.fori_loop(0, epw // _CH, chunk, 0)
    plsc.subcore_barrier()
    pltpu.sync_copy(acc.at[pl.ds(s * npr, npr)],
                    out_r.at[c, pl.ds(s * npr, npr)])


def _deg_pass(dst3d, wpad, zeros_1):
    return pl.kernel(
        _deg_body,
        out_type=jax.ShapeDtypeStruct((_NC, _NP), jnp.float32),
        mesh=_sc_mesh(),
        compiler_params=pltpu.CompilerParams(needs_layout_passes=False),
        scratch_types=[
            pltpu.VMEM((_SUB, 128), jnp.int32),
            pltpu.VMEM((_CH,), jnp.float32),
            pltpu.VMEM_SHARED((_NP,), jnp.float32),
        ],
    )(dst3d, wpad, zeros_1)


def _p3_body(src_r, dst_r, w_r, tab_r, zeros_r, out_r,
             srcv, dstv, wv, valsv, tabv, acc):
    c = lax.axis_index("c")
    s = lax.axis_index("s")
    npr = _NP // _NS
    pltpu.sync_copy(zeros_r.at[pl.ds(s * npr, npr)], acc.at[pl.ds(s * npr, npr)])
    pltpu.sync_copy(tab_r, tabv)
    plsc.subcore_barrier()
    epw = _EP // (_NC * _NS)
    wid = s * _NC + c
    rr0 = wid * (epw // _CH)

    def chunk(i, car):
        rr = rr0 + i
        pltpu.sync_copy(src_r.at[pl.ds(rr * _CH, _CH)], srcv)
        pltpu.sync_copy(dst_r.at[rr], dstv)
        pltpu.sync_copy(w_r.at[pl.ds(rr * _CH, _CH)], wv)

        def grp(g, car2):
            sl = pl.ds(g * _L, _L)
            vals = plsc.load_gather(tabv, [srcv[sl]]) * wv[sl]
            valsv[sl] = vals
            return car2

        lax.fori_loop(0, _CH // _L, grp, 0, unroll=4)
        for j in range(_SUB):
            pltpu.sync_copy(valsv.at[pl.ds(j * 128, 128)],
                            acc.at[dstv.at[j]], add=True)
        return car

    lax.fori_loop(0, epw // _CH, chunk, 0)
    plsc.subcore_barrier()
    pltpu.sync_copy(acc.at[pl.ds(s * npr, npr)],
                    out_r.at[c, pl.ds(s * npr, npr)])


def _p3_pass(table1, src1d, dst3d, wpad, zeros_1):
    return pl.kernel(
        _p3_body,
        out_type=jax.ShapeDtypeStruct((_NC, _NP), jnp.float32),
        mesh=_sc_mesh(),
        compiler_params=pltpu.CompilerParams(needs_layout_passes=False),
        scratch_types=[
            pltpu.VMEM((_CH,), jnp.int32),
            pltpu.VMEM((_SUB, 128), jnp.int32),
            pltpu.VMEM((_CH,), jnp.float32),
            pltpu.VMEM((_CH,), jnp.float32),
            pltpu.VMEM((_NP,), jnp.float32),
            pltpu.VMEM_SHARED((_NP,), jnp.float32),
        ],
    )(src1d, dst3d, wpad, table1, zeros_1)


def _stage_a_body(deg0_ref, deg1_ref, x_ref, h0_ref, zst_ref, dinv_ref):
    h = pl.program_id(0)
    dinv = lax.rsqrt(deg0_ref[...] + deg1_ref[...] + 1.0)
    dinv_ref[...] = dinv

    @pl.when(h == 0)
    def _():
        zst_ref[...] = x_ref[...] * dinv

    @pl.when(h == 1)
    def _():
        zst_ref[...] = h0_ref[...] * dinv


def _stage_b_body(p1a_ref, p1b_ref, x_ref, h0_ref, c0_ref, dinv_ref, catp_ref,
                  wz_ref, wci_ref, wcf_ref, wco_ref, bi_ref, bf_ref, bc_ref, bo_ref,
                  wfc1a_ref, wfc1b_ref,
                  gh_ref, beh_ref, gc_ref, bec_ref, go_ref, beo_ref,
                  hid_ref, cell_ref, u1_ref, u1s_ref):
    dinv = dinv_ref[...]
    x = x_ref[...]
    h0 = h0_ref[...]
    c0 = c0_ref[...]
    d2 = dinv * dinv
    za = jnp.concatenate(
        [dinv * p1a_ref[...] + d2 * x,
         dinv * p1b_ref[...] + d2 * h0], axis=1)
    g = jnp.dot(za, wz_ref[...], preferred_element_type=jnp.float32)
    I = jax.nn.sigmoid(g[:, 0 * HID:1 * HID] + wci_ref[...] * c0 + bi_ref[...])
    F = jax.nn.sigmoid(g[:, 1 * HID:2 * HID] + wcf_ref[...] * c0 + bf_ref[...])
    T = jnp.tanh(g[:, 2 * HID:3 * HID] + bc_ref[...])
    c_new = F * c0 + I * T
    O = jax.nn.sigmoid(g[:, 3 * HID:4 * HID] + wco_ref[...] * c_new + bo_ref[...])
    h_new = O * jnp.tanh(c_new)

    def ln(v, gg, bb):
        m = jnp.mean(v, axis=-1, keepdims=True)
        vv = jnp.mean((v - m) * (v - m), axis=-1, keepdims=True)
        return (v - m) * lax.rsqrt(vv + 1e-5) * gg + bb

    hid_ref[...] = ln(h_new, gh_ref[...], beh_ref[...])
    cell_ref[...] = ln(c_new, gc_ref[...], bec_ref[...])
    out = jnp.maximum(ln(h_new, go_ref[...], beo_ref[...]), 0.0)
    u1 = (jnp.dot(out, wfc1a_ref[...], preferred_element_type=jnp.float32)
          + jnp.dot(catp_ref[...], wfc1b_ref[...], preferred_element_type=jnp.float32))
    u1_ref[...] = u1
    u1s_ref[...] = u1 * dinv


def _stage_c_body(p2a_ref, p2b_ref, u1_ref, dinv_ref, bfc1_ref, w2_ref,
                  u2_ref, u2s_ref):
    dinv = dinv_ref[...]
    a1 = (dinv * (p2a_ref[...] + p2b_ref[...])
          + dinv * dinv * u1_ref[...] + bfc1_ref[...])
    r = jnp.maximum(a1, 0.0)
    u2 = jnp.sum(r * w2_ref[...], axis=-1, keepdims=True)
    u2_ref[...] = u2
    u2s_ref[...] = u2 * dinv


def _stage_d_body(p30_ref, p31_ref, u2_ref, dinv_ref, x0_ref, bfc2_ref, out_ref):
    dinv = dinv_ref[...]
    v = (dinv * (p30_ref[...] + p31_ref[...])
         + dinv * dinv * u2_ref[...] + bfc2_ref[...])
    out_ref[...] = jnp.tanh(v) + x0_ref[...]


def _row_spec(w):
    return pl.BlockSpec((_BB, w), lambda i: (i, 0))


def _bcast_spec(shape):
    return pl.BlockSpec(shape, lambda i: tuple(0 for _ in shape))


def kernel(X, edge_index, edge_weight, concat_layers, H, C,
           Wxi, Whi, wci, b_i, Wxf, Whf, wcf, b_f, Wxc, Whc, b_c,
           Wxo, Who, wco, b_o, W_fc1, b_fc1, W_fc2, b_fc2,
           g_o, be_o, g_h, be_h, g_c, be_c):
    h0 = H[0]
    c0 = C[0]
    grid = (N // _BB,)
    nb = N // _BB

    src1d = jnp.concatenate(
        [edge_index[0], jnp.zeros((_EP - E,), jnp.int32)])
    dst1d = jnp.concatenate(
        [edge_index[1], jnp.zeros((_EP - E,), jnp.int32)])
    wpad = jnp.concatenate(
        [edge_weight, jnp.zeros((_EP - E,), jnp.float32)])
    src3d = src1d.reshape(_EP // _CH, _SUB, 128)
    dst3d = dst1d.reshape(_EP // _CH, _SUB, 128)
    zeros_w = jnp.zeros((_NP, 128), jnp.float32)
    zeros_1 = jnp.zeros((_NP,), jnp.float32)

    deg2 = _deg_pass(dst3d, wpad, zeros_1)
    deg0 = deg2[0, :N, None]
    deg1 = deg2[1, :N, None]

    zst, dinv = pl.pallas_call(
        _stage_a_body,
        grid=(2, nb),
        in_specs=[pl.BlockSpec((_BB, 1), lambda h, i: (i, 0)),
                  pl.BlockSpec((_BB, 1), lambda h, i: (i, 0)),
                  pl.BlockSpec((_BB, HID), lambda h, i: (i, 0)),
                  pl.BlockSpec((_BB, HID), lambda h, i: (i, 0))],
        out_specs=[pl.BlockSpec((_BB, HID), lambda h, i: (h * (N // _BB) + i, 0)),
                   pl.BlockSpec((_BB, 1), lambda h, i: (i, 0))],
        out_shape=[jax.ShapeDtypeStruct((2 * N, HID), jnp.float32),
                   jax.ShapeDtypeStruct((N, 1), jnp.float32)],
    )(deg0, deg1, X, h0)

    p1 = _agg_wide(zst, src3d, dst3d, wpad, zeros_w, True)

    wz = jnp.concatenate([
        jnp.concatenate([Wxi, Wxf, Wxc, Wxo], axis=1),
        jnp.concatenate([Whi, Whf, Whc, Who], axis=1)], axis=0)
    catp = jnp.pad(concat_layers, ((0, 0), (0, 5)))
    wfc1a = W_fc1[:HID]
    wfc1b = jnp.pad(W_fc1[HID:], ((0, 5), (0, 0)))

    hidden, cell, u1, u1s = pl.pallas_call(
        _stage_b_body,
        grid=grid,
        in_specs=[_row_spec(HID), _row_spec(HID), _row_spec(HID), _row_spec(HID),
                  _row_spec(HID), _row_spec(1), _row_spec(8),
                  _bcast_spec((2 * HID, 4 * HID)),
                  _bcast_spec((1, HID)), _bcast_spec((1, HID)), _bcast_spec((1, HID)),
                  _bcast_spec((1, HID)), _bcast_spec((1, HID)), _bcast_spec((1, HID)),
                  _bcast_spec((1, HID)),
                  _bcast_spec((HID, HID)), _bcast_spec((8, HID)),
                  _bcast_spec((1, HID)), _bcast_spec((1, HID)), _bcast_spec((1, HID)),
                  _bcast_spec((1, HID)), _bcast_spec((1, HID)), _bcast_spec((1, HID))],
        out_specs=[_row_spec(HID)] * 4,
        out_shape=[jax.ShapeDtypeStruct((N, HID), jnp.float32)] * 4,
    )(p1[0], p1[1], X, h0, c0, dinv, catp, wz,
      wci, wcf, wco, b_i, b_f, b_c, b_o,
      wfc1a, wfc1b,
      g_h[None, :], be_h[None, :], g_c[None, :], be_c[None, :],
      g_o[None, :], be_o[None, :])

    p2 = _agg_wide(u1s, src3d, dst3d, wpad, zeros_w, False)

    u2, u2s = pl.pallas_call(
        _stage_c_body,
        grid=grid,
        in_specs=[_row_spec(HID), _row_spec(HID), _row_spec(HID), _row_spec(1),
                  _bcast_spec((1, HID)), _bcast_spec((1, HID))],
        out_specs=[_row_spec(1), _row_spec(1)],
        out_shape=[jax.ShapeDtypeStruct((N, 1), jnp.float32)] * 2,
    )(p2[0], p2[1], u1, dinv, b_fc1[None, :], W_fc2.T)

    u2s_pad = jnp.pad(u2s[:, 0], (0, _NP - N))
    p3 = _p3_pass(u2s_pad, src1d, dst3d, wpad, zeros_1)

    out = pl.pallas_call(
        _stage_d_body,
        grid=grid,
        in_specs=[_row_spec(1), _row_spec(1), _row_spec(1), _row_spec(1),
                  _row_spec(1), _bcast_spec((1, 1))],
        out_specs=_row_spec(1),
        out_shape=jax.ShapeDtypeStruct((N, 1), jnp.float32),
    )(p3[0, :N, None], p3[1, :N, None], u2, dinv, X[:, 0:1], b_fc2[None, :])

    return (out, hidden[None], cell[None])


# speedup vs baseline: 19.1322x; 1.2088x over previous
import functools

import jax
import jax.numpy as jnp
from jax import lax
from jax.experimental import pallas as pl
from jax.experimental.pallas import tpu as pltpu
from jax.experimental.pallas import tpu_sc as plsc

N = 10000
F_IN = 128
HID = 128
E = 320000

_NC, _NS, _L = 2, 16, 16
_EP = 327680
_NP = 10240
_CH = 1024
_SUB = _CH // 128

_BB = 400


def _sc_mesh():
    return plsc.VectorSubcoreMesh(core_axis_name="c", subcore_axis_name="s")


def _agg_wide_body(feature_split, src_r, dst_r, w_r, tab_r, zeros_r, out_r,
                   idxv, dstv, wv, rows, acc, sem0, sem1):
    c = lax.axis_index("c")
    s = lax.axis_index("s")
    npr = _NP // _NS
    pltpu.sync_copy(zeros_r.at[pl.ds(s * npr, npr)], acc.at[pl.ds(s * npr, npr)])
    plsc.subcore_barrier()
    if feature_split:
        epw = _EP // _NS
        rr0 = s * (epw // _CH)
        coff = c * N
    else:
        epw = _EP // (_NC * _NS)
        wid = s * _NC + c
        rr0 = wid * (epw // _CH)
        coff = 0

    def chunk(i, car):
        rr = rr0 + i
        pltpu.sync_copy(src_r.at[rr], idxv)
        pltpu.sync_copy(dst_r.at[rr], dstv)
        pltpu.sync_copy(w_r.at[pl.ds(rr * _CH, _CH)], wv)
        if feature_split:
            for j in range(_SUB):
                for k in range(128 // _L):
                    sl = pl.ds(k * _L, _L)
                    idxv[j, sl] = idxv[j, sl] + coff
        pltpu.async_copy(tab_r.at[idxv.at[0]], rows.at[0], sem0)
        for j in range(_SUB):
            b = j % 2
            if j + 1 < _SUB:
                pltpu.async_copy(tab_r.at[idxv.at[j + 1]], rows.at[1 - b],
                                 sem0 if (j + 1) % 2 == 0 else sem1)
            pltpu.make_async_copy(tab_r.at[idxv.at[j]], rows.at[b],
                                  sem0 if b == 0 else sem1).wait()

            def scale(e, car2):
                wb = plsc.load_gather(wv, [jnp.broadcast_to(j * 128 + e, (_L,))])
                for q in range(128 // _L):
                    sl = pl.ds(q * _L, _L)
                    rows[b, e, sl] = rows[b, e, sl] * wb
                return car2

            lax.fori_loop(0, 128, scale, 0, unroll=4)
            pltpu.sync_copy(rows.at[b], acc.at[dstv.at[j]], add=True)
        return car

    lax.fori_loop(0, epw // _CH, chunk, 0)
    plsc.subcore_barrier()
    pltpu.sync_copy(acc.at[pl.ds(s * npr, npr)],
                    out_r.at[c, pl.ds(s * npr, npr)])


def _agg_wide(table, src3d, dst3d, wpad, zeros_w, feature_split):
    body = functools.partial(_agg_wide_body, feature_split)
    return pl.kernel(
        body,
        out_type=jax.ShapeDtypeStruct((_NC, _NP, 128), jnp.float32),
        mesh=_sc_mesh(),
        compiler_params=pltpu.CompilerParams(needs_layout_passes=False),
        scratch_types=[
            pltpu.VMEM((_SUB, 128), jnp.int32),
            pltpu.VMEM((_SUB, 128), jnp.int32),
            pltpu.VMEM((_CH,), jnp.float32),
            pltpu.VMEM((2, 128, 128), jnp.float32),
            pltpu.VMEM_SHARED((_NP, 128), jnp.float32),
            pltpu.SemaphoreType.DMA,
            pltpu.SemaphoreType.DMA,
        ],
    )(src3d, dst3d, wpad, table, zeros_w)


def _deg_body(dst_r, w_r, zeros_r, out_r, dstv, wv, acc):
    c = lax.axis_index("c")
    s = lax.axis_index("s")
    npr = _NP // _NS
    pltpu.sync_copy(zeros_r.at[pl.ds(s * npr, npr)], acc.at[pl.ds(s * npr, npr)])
    plsc.subcore_barrier()
    epw = _EP // (_NC * _NS)
    wid = s * _NC + c
    rr0 = wid * (epw // _CH)

    def chunk(i, car):
        rr = rr0 + i
        pltpu.sync_copy(dst_r.at[rr], dstv)
        pltpu.sync_copy(w_r.at[pl.ds(rr * _CH, _CH)], wv)
        for j in range(_SUB):
            pltpu.sync_copy(wv.at[pl.ds(j * 128, 128)],
                            acc.at[dstv.at[j]], add=True)
        return car

    lax.fori_loop(0, epw // _CH, chunk, 0)
    plsc.subcore_barrier()
    pltpu.sync_copy(acc.at[pl.ds(s * npr, npr)],
                    out_r.at[c, pl.ds(s * npr, npr)])


def _deg_pass(dst3d, wpad, zeros_1):
    return pl.kernel(
        _deg_body,
        out_type=jax.ShapeDtypeStruct((_NC, _NP), jnp.float32),
        mesh=_sc_mesh(),
        compiler_params=pltpu.CompilerParams(needs_layout_passes=False),
        scratch_types=[
            pltpu.VMEM((_SUB, 128), jnp.int32),
            pltpu.VMEM((_CH,), jnp.float32),
            pltpu.VMEM_SHARED((_NP,), jnp.float32),
        ],
    )(dst3d, wpad, zeros_1)


def _p3_body(src_r, dst_r, w_r, tab_r, zeros_r, out_r,
             srcv, dstv, wv, valsv, tabv, acc):
    c = lax.axis_index("c")
    s = lax.axis_index("s")
    npr = _NP // _NS
    pltpu.sync_copy(zeros_r.at[pl.ds(s * npr, npr)], acc.at[pl.ds(s * npr, npr)])
    pltpu.sync_copy(tab_r, tabv)
    plsc.subcore_barrier()
    epw = _EP // (_NC * _NS)
    wid = s * _NC + c
    rr0 = wid * (epw // _CH)

    def chunk(i, car):
        rr = rr0 + i
        pltpu.sync_copy(src_r.at[pl.ds(rr * _CH, _CH)], srcv)
        pltpu.sync_copy(dst_r.at[rr], dstv)
        pltpu.sync_copy(w_r.at[pl.ds(rr * _CH, _CH)], wv)

        def grp(g, car2):
            sl = pl.ds(g * _L, _L)
            vals = plsc.load_gather(tabv, [srcv[sl]]) * wv[sl]
            valsv[sl] = vals
            return car2

        lax.fori_loop(0, _CH // _L, grp, 0, unroll=4)
        for j in range(_SUB):
            pltpu.sync_copy(valsv.at[pl.ds(j * 128, 128)],
                            acc.at[dstv.at[j]], add=True)
        return car

    lax.fori_loop(0, epw // _CH, chunk, 0)
    plsc.subcore_barrier()
    pltpu.sync_copy(acc.at[pl.ds(s * npr, npr)],
                    out_r.at[c, pl.ds(s * npr, npr)])


def _p3_pass(table1, src1d, dst3d, wpad, zeros_1):
    return pl.kernel(
        _p3_body,
        out_type=jax.ShapeDtypeStruct((_NC, _NP), jnp.float32),
        mesh=_sc_mesh(),
        compiler_params=pltpu.CompilerParams(needs_layout_passes=False),
        scratch_types=[
            pltpu.VMEM((_CH,), jnp.int32),
            pltpu.VMEM((_SUB, 128), jnp.int32),
            pltpu.VMEM((_CH,), jnp.float32),
            pltpu.VMEM((_CH,), jnp.float32),
            pltpu.VMEM((_NP,), jnp.float32),
            pltpu.VMEM_SHARED((_NP,), jnp.float32),
        ],
    )(src1d, dst3d, wpad, table1, zeros_1)


def _stage_a_body(deg0_ref, deg1_ref, x_ref, h0_ref, zst_ref, dinv_ref):
    h = pl.program_id(0)
    dinv = lax.rsqrt(deg0_ref[...] + deg1_ref[...] + 1.0)
    dinv_ref[...] = dinv

    @pl.when(h == 0)
    def _():
        zst_ref[...] = x_ref[...] * dinv

    @pl.when(h == 1)
    def _():
        zst_ref[...] = h0_ref[...] * dinv


def _stage_b_body(p1a_ref, p1b_ref, x_ref, h0_ref, c0_ref, dinv_ref, catp_ref,
                  wz_ref, wci_ref, wcf_ref, wco_ref, bi_ref, bf_ref, bc_ref, bo_ref,
                  wfc1a_ref, wfc1b_ref,
                  gh_ref, beh_ref, gc_ref, bec_ref, go_ref, beo_ref,
                  hid_ref, cell_ref, u1_ref, u1s_ref):
    dinv = dinv_ref[...]
    x = x_ref[...]
    h0 = h0_ref[...]
    c0 = c0_ref[...]
    d2 = dinv * dinv
    za = jnp.concatenate(
        [dinv * p1a_ref[...] + d2 * x,
         dinv * p1b_ref[...] + d2 * h0], axis=1)
    g = jnp.dot(za, wz_ref[...], preferred_element_type=jnp.float32)
    I = jax.nn.sigmoid(g[:, 0 * HID:1 * HID] + wci_ref[...] * c0 + bi_ref[...])
    F = jax.nn.sigmoid(g[:, 1 * HID:2 * HID] + wcf_ref[...] * c0 + bf_ref[...])
    T = jnp.tanh(g[:, 2 * HID:3 * HID] + bc_ref[...])
    c_new = F * c0 + I * T
    O = jax.nn.sigmoid(g[:, 3 * HID:4 * HID] + wco_ref[...] * c_new + bo_ref[...])
    h_new = O * jnp.tanh(c_new)

    def ln(v, gg, bb):
        m = jnp.mean(v, axis=-1, keepdims=True)
        vv = jnp.mean((v - m) * (v - m), axis=-1, keepdims=True)
        return (v - m) * lax.rsqrt(vv + 1e-5) * gg + bb

    hid_ref[...] = ln(h_new, gh_ref[...], beh_ref[...])
    cell_ref[...] = ln(c_new, gc_ref[...], bec_ref[...])
    out = jnp.maximum(ln(h_new, go_ref[...], beo_ref[...]), 0.0)
    u1 = (jnp.dot(out, wfc1a_ref[...], preferred_element_type=jnp.float32)
          + jnp.dot(catp_ref[...], wfc1b_ref[...], preferred_element_type=jnp.float32))
    u1_ref[...] = u1
    u1s_ref[...] = u1 * dinv


def _stage_c_body(p2a_ref, p2b_ref, u1_ref, dinv_ref, bfc1_ref, w2_ref,
                  u2_ref, u2s_ref):
    dinv = dinv_ref[...]
    a1 = (dinv * (p2a_ref[...] + p2b_ref[...])
          + dinv * dinv * u1_ref[...] + bfc1_ref[...])
    r = jnp.maximum(a1, 0.0)
    u2 = jnp.sum(r * w2_ref[...], axis=-1, keepdims=True)
    u2_ref[...] = u2
    u2s_ref[...] = u2 * dinv


def _stage_d_body(p30_ref, p31_ref, u2_ref, dinv_ref, x0_ref, bfc2_ref, out_ref):
    dinv = dinv_ref[...]
    v = (dinv * (p30_ref[...] + p31_ref[...])
         + dinv * dinv * u2_ref[...] + bfc2_ref[...])
    out_ref[...] = jnp.tanh(v) + x0_ref[...]


def _row_spec(w):
    return pl.BlockSpec((_BB, w), lambda i: (i, 0))


def _bcast_spec(shape):
    return pl.BlockSpec(shape, lambda i: tuple(0 for _ in shape))


def kernel(X, edge_index, edge_weight, concat_layers, H, C,
           Wxi, Whi, wci, b_i, Wxf, Whf, wcf, b_f, Wxc, Whc, b_c,
           Wxo, Who, wco, b_o, W_fc1, b_fc1, W_fc2, b_fc2,
           g_o, be_o, g_h, be_h, g_c, be_c):
    h0 = H[0]
    c0 = C[0]
    grid = (N // _BB,)
    nb = N // _BB

    src1d = jnp.concatenate(
        [edge_index[0], jnp.zeros((_EP - E,), jnp.int32)])
    dst1d = jnp.concatenate(
        [edge_index[1], jnp.zeros((_EP - E,), jnp.int32)])
    wpad = jnp.concatenate(
        [edge_weight, jnp.zeros((_EP - E,), jnp.float32)])
    src3d = src1d.reshape(_EP // _CH, _SUB, 128)
    dst3d = dst1d.reshape(_EP // _CH, _SUB, 128)
    zeros_w = jnp.zeros((_NP, 128), jnp.float32)
    zeros_1 = jnp.zeros((_NP,), jnp.float32)

    deg2 = _deg_pass(dst3d, wpad, zeros_1)
    deg0 = deg2[0, :N, None]
    deg1 = deg2[1, :N, None]

    zst, dinv = pl.pallas_call(
        _stage_a_body,
        grid=(2, nb),
        in_specs=[pl.BlockSpec((_BB, 1), lambda h, i: (i, 0)),
                  pl.BlockSpec((_BB, 1), lambda h, i: (i, 0)),
                  pl.BlockSpec((_BB, HID), lambda h, i: (i, 0)),
                  pl.BlockSpec((_BB, HID), lambda h, i: (i, 0))],
        out_specs=[pl.BlockSpec((_BB, HID), lambda h, i: (h * (N // _BB) + i, 0)),
                   pl.BlockSpec((_BB, 1), lambda h, i: (i, 0))],
        out_shape=[jax.ShapeDtypeStruct((2 * N, HID), jnp.float32),
                   jax.ShapeDtypeStruct((N, 1), jnp.float32)],
    )(deg0, deg1, X, h0)

    p1 = _agg_wide(zst, src3d, dst3d, wpad, zeros_w, True)

    wz = jnp.concatenate([
        jnp.concatenate([Wxi, Wxf, Wxc, Wxo], axis=1),
        jnp.concatenate([Whi, Whf, Whc, Who], axis=1)], axis=0)
    catp = jnp.pad(concat_layers, ((0, 0), (0, 5)))
    wfc1a = W_fc1[:HID]
    wfc1b = jnp.pad(W_fc1[HID:], ((0, 5), (0, 0)))

    hidden, cell, u1, u1s = pl.pallas_call(
        _stage_b_body,
        grid=grid,
        in_specs=[_row_spec(HID), _row_spec(HID), _row_spec(HID), _row_spec(HID),
                  _row_spec(HID), _row_spec(1), _row_spec(8),
                  _bcast_spec((2 * HID, 4 * HID)),
                  _bcast_spec((1, HID)), _bcast_spec((1, HID)), _bcast_spec((1, HID)),
                  _bcast_spec((1, HID)), _bcast_spec((1, HID)), _bcast_spec((1, HID)),
                  _bcast_spec((1, HID)),
                  _bcast_spec((HID, HID)), _bcast_spec((8, HID)),
                  _bcast_spec((1, HID)), _bcast_spec((1, HID)), _bcast_spec((1, HID)),
                  _bcast_spec((1, HID)), _bcast_spec((1, HID)), _bcast_spec((1, HID))],
        out_specs=[_row_spec(HID)] * 4,
        out_shape=[jax.ShapeDtypeStruct((N, HID), jnp.float32)] * 4,
    )(p1[0], p1[1], X, h0, c0, dinv, catp, wz,
      wci, wcf, wco, b_i, b_f, b_c, b_o,
      wfc1a, wfc1b,
      g_h[None, :], be_h[None, :], g_c[None, :], be_c[None, :],
      g_o[None, :], be_o[None, :])

    p2 = _agg_wide(u1s, src3d, dst3d, wpad, zeros_w, False)

    u2, u2s = pl.pallas_call(
        _stage_c_body,
        grid=grid,
        in_specs=[_row_spec(HID), _row_spec(HID), _row_spec(HID), _row_spec(1),
                  _bcast_spec((1, HID)), _bcast_spec((1, HID))],
        out_specs=[_row_spec(1), _row_spec(1)],
        out_shape=[jax.ShapeDtypeStruct((N, 1), jnp.float32)] * 2,
    )(p2[0], p2[1], u1, dinv, b_fc1[None, :], W_fc2.T)

    u2s_pad = jnp.pad(u2s[:, 0], (0, _NP - N))
    p3 = _p3_pass(u2s_pad, src1d, dst3d, wpad, zeros_1)

    out = pl.pallas_call(
        _stage_d_body,
        grid=grid,
        in_specs=[_row_spec(1), _row_spec(1), _row_spec(1), _row_spec(1),
                  _row_spec(1), _bcast_spec((1, 1))],
        out_specs=_row_spec(1),
        out_shape=jax.ShapeDtypeStruct((N, 1), jnp.float32),
    )(p3[0, :N, None], p3[1, :N, None], u2, dinv, X[:, 0:1], b_fc2[None, :])

    return (out, hidden[None], cell[None])
